# baseline (speedup 1.0000x reference)
import functools

import jax
import jax.numpy as jnp
from jax import lax
from jax.experimental import pallas as pl
from jax.experimental.pallas import tpu as pltpu
from jax.experimental.pallas import tpu_sc as plsc

NW = 32
LANES = 128
L = 16


def _sc_gather_main_t(idx2d, cell_means, B, D, DM):
    b_per_w = B // NW
    n_chunks = DM // LANES

    mesh = plsc.VectorSubcoreMesh(core_axis_name="c", subcore_axis_name="s")

    @functools.partial(
        pl.kernel,
        mesh=mesh,
        out_type=jax.ShapeDtypeStruct((D, B), jnp.float32),
        compiler_params=pltpu.CompilerParams(needs_layout_passes=False),
        scratch_types=[
            pltpu.VMEM((NW, b_per_w), jnp.int32),
            *[pltpu.VMEM((b_per_w, LANES), jnp.float32) for _ in range(2)],
            *[pltpu.VMEM((LANES, b_per_w), jnp.float32) for _ in range(2)],
            *[pltpu.SemaphoreType.DMA for _ in range(4)],
        ],
    )
    def gather_kernel(idx_hbm, table_hbm, outt_hbm, idx_v, *rest):
        ibufs = rest[0:2]
        obufs = rest[2:4]
        gsems = rest[4:6]
        wsems = rest[6:8]
        wid = lax.axis_index("s") * 2 + lax.axis_index("c")
        col0 = wid * b_per_w
        pltpu.sync_copy(idx_hbm, idx_v)

        def g_start(c, p):
            pltpu.async_copy(
                table_hbm.at[idx_v.at[wid], pl.ds(c * LANES, LANES)],
                ibufs[p],
                gsems[p],
            )

        def g_wait(c, p):
            pltpu.make_async_copy(
                table_hbm.at[idx_v.at[wid], pl.ds(c * LANES, LANES)],
                ibufs[p],
                gsems[p],
            ).wait()

        def out_at(c):
            return outt_hbm.at[pl.ds(c * LANES, LANES), pl.ds(col0, b_per_w)]

        def w_start(c, p):
            pltpu.async_copy(obufs[p], out_at(c), wsems[p])

        def w_wait(c, p):
            pltpu.make_async_copy(obufs[p], out_at(c), wsems[p]).wait()

        iota16 = lax.iota(jnp.int32, L)

        def transpose(p):
            def tbody(g, carry):
                cols = jnp.full((L,), g, jnp.int32)
                for j in range(b_per_w // L):
                    rows = iota16 + (j * L)
                    v = plsc.load_gather(ibufs[p], [rows, cols])
                    obufs[p][g, pl.ds(j * L, L)] = v
                return carry

            lax.fori_loop(0, LANES, tbody, 0, unroll=8)

        g_start(0, 0)

        def body(c2, carry):
            for p in range(2):
                c = c2 * 2 + p

                @pl.when(c + 1 < n_chunks)
                def _():
                    g_wait(c, p)
                    g_start(c + 1, 1 - p)

                @pl.when(c + 1 >= n_chunks)
                def _():
                    g_wait(c, p)

                @pl.when(c >= 2)
                def _():
                    w_wait(c - 2, p)

                transpose(p)
                w_start(c, p)
            return carry

        lax.fori_loop(0, n_chunks // 2, body, 0)
        w_wait(n_chunks - 2, 0)
        w_wait(n_chunks - 1, 1)

    return gather_kernel(idx2d, cell_means)


def _tc_gather_tail_t(idx, cell_means, outt_main, B, V, D, DM):
    DT = D - DM

    def tail_kernel(idx_ref, tab_ref, _, o_ref):
        onehot = (
            lax.broadcasted_iota(jnp.int32, (V, B), 0) == idx_ref[...]
        ).astype(jnp.float32)
        res = lax.dot_general(
            tab_ref[...],
            onehot,
            (((0,), (0,)), ((), ())),
            preferred_element_type=jnp.float32,
            precision=lax.Precision.HIGHEST,
        )
        o_ref[...] = res[:DT, :]

    return pl.pallas_call(
        tail_kernel,
        grid=(1,),
        in_specs=[
            pl.BlockSpec((1, B), lambda g: (0, 0)),
            pl.BlockSpec((V, LANES), lambda g: (0, DM // LANES)),
            pl.BlockSpec(memory_space=pl.ANY),
        ],
        out_specs=pl.BlockSpec((DT, B), lambda g: (DM // DT, 0)),
        out_shape=jax.ShapeDtypeStruct((D, B), jnp.float32),
        input_output_aliases={2: 0},
    )(idx.reshape(1, B), cell_means, outt_main)


def kernel(cell_index, cell_means):
    B = cell_index.shape[0]
    V, D = cell_means.shape
    DM = (D // LANES) * LANES

    idx = cell_index.astype(jnp.int32)
    idx2d = idx.reshape(NW, B // NW)

    outt = _sc_gather_main_t(idx2d, cell_means, B, D, DM)
    if DM != D:
        outt = _tc_gather_tail_t(idx, cell_means, outt, B, V, D, DM)
    return outt.T


# speedup vs baseline: 2.2546x; 2.2546x over previous
import functools

import jax
import jax.numpy as jnp
from jax import lax
from jax.experimental import pallas as pl
from jax.experimental.pallas import tpu as pltpu
from jax.experimental.pallas import tpu_sc as plsc

NW = 32
LANES = 128
L = 16


def _sc_gather_main_t(idx2d, cell_means, B, D, DM):
    b_per_w = B // NW
    n_chunks = DM // LANES

    mesh = plsc.VectorSubcoreMesh(core_axis_name="c", subcore_axis_name="s")

    @functools.partial(
        pl.kernel,
        mesh=mesh,
        out_type=jax.ShapeDtypeStruct((D, B), jnp.float32),
        compiler_params=pltpu.CompilerParams(needs_layout_passes=False),
        scratch_types=[
            pltpu.VMEM((NW, b_per_w), jnp.int32),
            *[pltpu.VMEM((b_per_w, LANES), jnp.float32) for _ in range(2)],
            *[pltpu.VMEM((LANES, b_per_w), jnp.float32) for _ in range(2)],
            *[pltpu.SemaphoreType.DMA for _ in range(4)],
        ],
    )
    def gather_kernel(idx_hbm, table_hbm, outt_hbm, idx_v, *rest):
        ibufs = rest[0:2]
        obufs = rest[2:4]
        gsems = rest[4:6]
        wsems = rest[6:8]
        wid = lax.axis_index("s") * 2 + lax.axis_index("c")
        col0 = wid * b_per_w
        pltpu.sync_copy(idx_hbm, idx_v)

        def g_start(c, p):
            pltpu.async_copy(
                table_hbm.at[idx_v.at[wid], pl.ds(c * LANES, LANES)],
                ibufs[p],
                gsems[p],
            )

        def g_wait(c, p):
            pltpu.make_async_copy(
                table_hbm.at[idx_v.at[wid], pl.ds(c * LANES, LANES)],
                ibufs[p],
                gsems[p],
            ).wait()

        def out_at(c):
            return outt_hbm.at[pl.ds(c * LANES, LANES), pl.ds(col0, b_per_w)]

        def w_start(c, p):
            pltpu.async_copy(obufs[p], out_at(c), wsems[p])

        def w_wait(c, p):
            pltpu.make_async_copy(obufs[p], out_at(c), wsems[p]).wait()

        iota16 = lax.iota(jnp.int32, L)
        perms = [(iota16 + m) & (L - 1) for m in range(L)]

        def transpose(p):
            def jbody(j, carry):
                rows = iota16 + (j * L)
                for t in range(LANES // L):
                    for m in range(L):
                        cols = perms[m] + (t * L)
                        v = plsc.load_gather(ibufs[p], [rows, cols])
                        plsc.store_scatter(obufs[p], [cols, rows], v)
                return carry

            lax.fori_loop(0, b_per_w // L, jbody, 0)

        g_start(0, 0)

        def body(c2, carry):
            for p in range(2):
                c = c2 * 2 + p

                @pl.when(c + 1 < n_chunks)
                def _():
                    g_wait(c, p)
                    g_start(c + 1, 1 - p)

                @pl.when(c + 1 >= n_chunks)
                def _():
                    g_wait(c, p)

                @pl.when(c >= 2)
                def _():
                    w_wait(c - 2, p)

                transpose(p)
                w_start(c, p)
            return carry

        lax.fori_loop(0, n_chunks // 2, body, 0)
        w_wait(n_chunks - 2, 0)
        w_wait(n_chunks - 1, 1)

    return gather_kernel(idx2d, cell_means)


def _tc_gather_tail_t(idx, cell_means, outt_main, B, V, D, DM):
    DT = D - DM

    def tail_kernel(idx_ref, tab_ref, _, o_ref):
        onehot = (
            lax.broadcasted_iota(jnp.int32, (V, B), 0) == idx_ref[...]
        ).astype(jnp.float32)
        res = lax.dot_general(
            tab_ref[...],
            onehot,
            (((0,), (0,)), ((), ())),
            preferred_element_type=jnp.float32,
            precision=lax.Precision.HIGHEST,
        )
        o_ref[...] = res[:DT, :]

    return pl.pallas_call(
        tail_kernel,
        grid=(1,),
        in_specs=[
            pl.BlockSpec((1, B), lambda g: (0, 0)),
            pl.BlockSpec((V, LANES), lambda g: (0, DM // LANES)),
            pl.BlockSpec(memory_space=pl.ANY),
        ],
        out_specs=pl.BlockSpec((DT, B), lambda g: (DM // DT, 0)),
        out_shape=jax.ShapeDtypeStruct((D, B), jnp.float32),
        input_output_aliases={2: 0},
    )(idx.reshape(1, B), cell_means, outt_main)


def kernel(cell_index, cell_means):
    B = cell_index.shape[0]
    V, D = cell_means.shape
    DM = (D // LANES) * LANES

    idx = cell_index.astype(jnp.int32)
    idx2d = idx.reshape(NW, B // NW)

    outt = _sc_gather_main_t(idx2d, cell_means, B, D, DM)
    if DM != D:
        outt = _tc_gather_tail_t(idx, cell_means, outt, B, V, D, DM)
    return outt.T
